# baseline (speedup 1.0000x reference)
import functools

import jax
import jax.numpy as jnp
from jax import lax
from jax.experimental import pallas as pl
from jax.experimental.pallas import tpu as pltpu, tpu_sc as plsc

N = 10000
E = 320000
D = 128
H = 32
C = 30
REG = 0.01

NC = 2
NS = 16
NW = NC * NS
K = 128
CH = 80
EPW = CH * K
EP = NW * EPW
RPS = 632
RP = NS * RPS
STEAL = 32
STEAL_FF = 48

_MESH = plsc.VectorSubcoreMesh(core_axis_name="c", subcore_axis_name="s")


@functools.partial(
    pl.kernel,
    out_type=jax.ShapeDtypeStruct((NC, RP, H), jnp.float32),
    mesh=_MESH,
    compiler_params=pltpu.CompilerParams(use_tc_tiling_on_sc=False),
    scratch_types=[
        pltpu.VMEM((CH, K), jnp.int32),
        pltpu.VMEM((CH, K), jnp.int32),
        pltpu.VMEM((K, H), jnp.float32),
        pltpu.VMEM((K, H), jnp.float32),
        pltpu.VMEM_SHARED((RP, H), jnp.float32),
        pltpu.SemaphoreType.DMA,
        pltpu.SemaphoreType.DMA,
    ],
)
def _sc_agg(hs_hbm, src_hbm, dst_hbm, zeros_hbm, out_hbm,
            srcv, dstv, rows0, rows1, acc, sem0, sem1):
    c = lax.axis_index("c")
    s = lax.axis_index("s")
    wid = s * NC + c
    row0 = pl.multiple_of(s * RPS, 8)
    pltpu.sync_copy(zeros_hbm.at[pl.ds(row0, RPS)], acc.at[pl.ds(row0, RPS)])
    plsc.subcore_barrier()
    pltpu.sync_copy(src_hbm.at[wid], srcv)
    pltpu.sync_copy(dst_hbm.at[wid], dstv)

    def pipe(cnt):
        pltpu.async_copy(hs_hbm.at[srcv.at[0]], rows0, sem0)
        pltpu.async_copy(hs_hbm.at[srcv.at[1]], rows1, sem1)

        def body(i, carry):
            g = 2 * i
            pltpu.make_async_copy(hs_hbm.at[srcv.at[g]], rows0, sem0).wait()
            pltpu.sync_copy(rows0, acc.at[dstv.at[g]], add=True)

            @pl.when(g + 2 < cnt)
            def _():
                pltpu.async_copy(hs_hbm.at[srcv.at[g + 2]], rows0, sem0)

            pltpu.make_async_copy(hs_hbm.at[srcv.at[g + 1]], rows1, sem1).wait()
            pltpu.sync_copy(rows1, acc.at[dstv.at[g + 1]], add=True)

            @pl.when(g + 3 < cnt)
            def _():
                pltpu.async_copy(hs_hbm.at[srcv.at[g + 3]], rows1, sem1)

            return carry

        lax.fori_loop(0, cnt // 2, body, 0)

    pw = s * NC + 1

    @pl.when(c == 0)
    def _():
        pipe(CH)
        pltpu.sync_copy(src_hbm.at[pw, pl.ds(CH - STEAL, STEAL)], srcv.at[pl.ds(0, STEAL)])
        pltpu.sync_copy(dst_hbm.at[pw, pl.ds(CH - STEAL, STEAL)], dstv.at[pl.ds(0, STEAL)])
        pipe(STEAL)

    @pl.when(c == 1)
    def _():
        pipe(CH - STEAL)

    plsc.subcore_barrier()
    pltpu.sync_copy(acc.at[pl.ds(row0, RPS)], out_hbm.at[c, pl.ds(row0, RPS)])


DW = 16


@functools.partial(
    pl.kernel,
    out_type=jax.ShapeDtypeStruct((NC, RP, DW), jnp.float32),
    mesh=_MESH,
    compiler_params=pltpu.CompilerParams(use_tc_tiling_on_sc=False),
    scratch_types=[
        pltpu.VMEM((CH, K), jnp.int32),
        pltpu.VMEM((K, DW), jnp.float32),
        pltpu.VMEM_SHARED((RP, DW), jnp.float32),
    ],
)
def _sc_deg(dst_hbm, ones_hbm, zeros_hbm, out_hbm, dstv, ones, acc):
    c = lax.axis_index("c")
    s = lax.axis_index("s")
    wid = s * NC + c
    row0 = pl.multiple_of(s * RPS, 8)
    pltpu.sync_copy(zeros_hbm.at[pl.ds(row0, RPS)], acc.at[pl.ds(row0, RPS)])
    pltpu.sync_copy(ones_hbm, ones)
    plsc.subcore_barrier()
    pltpu.sync_copy(dst_hbm.at[wid], dstv)

    def body(ci, carry):
        pltpu.sync_copy(ones, acc.at[dstv.at[ci]], add=True)
        return carry

    lax.fori_loop(0, CH, body, 0)
    plsc.subcore_barrier()
    pltpu.sync_copy(acc.at[pl.ds(row0, RPS)], out_hbm.at[c, pl.ds(row0, RPS)])


@functools.partial(
    pl.kernel,
    out_type=jax.ShapeDtypeStruct((NC, NS, 16), jnp.float32),
    mesh=_MESH,
    compiler_params=pltpu.CompilerParams(
        use_tc_tiling_on_sc=False, needs_layout_passes=False),
    scratch_types=[
        pltpu.VMEM((CH, K), jnp.int32),
        pltpu.VMEM((CH, K), jnp.int32),
        pltpu.VMEM((CH * K,), jnp.float32),
        pltpu.VMEM((K, H), jnp.float32),
        pltpu.VMEM((K, H), jnp.float32),
        pltpu.VMEM((K, H), jnp.float32),
        pltpu.VMEM((K, H), jnp.float32),
        pltpu.VMEM((16,), jnp.float32),
        pltpu.SemaphoreType.DMA,
        pltpu.SemaphoreType.DMA,
        pltpu.SemaphoreType.DMA,
        pltpu.SemaphoreType.DMA,
    ],
)
def _sc_ff(fx_hbm, src_hbm, dst_hbm, pred_hbm, out_hbm,
           srcv, dstv, predv, rs0, rd0, rs1, rd1, ssev,
           semS0, semD0, semS1, semD1):
    c = lax.axis_index("c")
    s = lax.axis_index("s")
    wid = s * NC + c
    pltpu.sync_copy(src_hbm.at[wid], srcv)
    pltpu.sync_copy(dst_hbm.at[wid], dstv)
    pltpu.sync_copy(pred_hbm.at[pl.ds(wid * EPW, EPW)], predv)

    def start(ci, rows_s, rows_d, sem_s, sem_d):
        pltpu.async_copy(fx_hbm.at[srcv.at[ci]], rows_s, sem_s)
        pltpu.async_copy(fx_hbm.at[dstv.at[ci]], rows_d, sem_d)

    def wait(ci, rows_s, rows_d, sem_s, sem_d):
        pltpu.make_async_copy(fx_hbm.at[srcv.at[ci]], rows_s, sem_s).wait()
        pltpu.make_async_copy(fx_hbm.at[dstv.at[ci]], rows_d, sem_d).wait()

    def chunk_dot(ci, rows_s, rows_d, sse):
        accs = [sse, 0.0, 0.0, 0.0]
        for e0 in range(0, K, 16):
            pv = predv[pl.ds(ci * K + e0, 16)]
            for j in range(16):
                e = e0 + j
                a0 = rows_s[e, pl.ds(0, 16)]
                a1 = rows_s[e, pl.ds(16, 16)]
                b0 = rows_d[e, pl.ds(0, 16)]
                b1 = rows_d[e, pl.ds(16, 16)]
                dot = jnp.sum(a0 * b0 + a1 * b1)
                err = dot - pv[j]
                accs[e % 4] += err * err
        return (accs[0] + accs[1]) + (accs[2] + accs[3])

    def pipe(cnt):
        start(0, rs0, rd0, semS0, semD0)
        start(1, rs1, rd1, semS1, semD1)

        def body(i, sse):
            g = 2 * i
            wait(g, rs0, rd0, semS0, semD0)
            sse = chunk_dot(g, rs0, rd0, sse)

            @pl.when(g + 2 < cnt)
            def _():
                start(g + 2, rs0, rd0, semS0, semD0)

            wait(g + 1, rs1, rd1, semS1, semD1)
            sse = chunk_dot(g + 1, rs1, rd1, sse)

            @pl.when(g + 3 < cnt)
            def _():
                start(g + 3, rs1, rd1, semS1, semD1)

            return sse

        sse = lax.fori_loop(0, cnt // 2, body, jnp.float32(0.0))
        ssev[...] = ssev[...] + jnp.full((16,), sse, jnp.float32)

    ssev[...] = jnp.zeros((16,), jnp.float32)
    pw = s * NC + 1

    @pl.when(c == 0)
    def _():
        pipe(CH)
        pltpu.sync_copy(src_hbm.at[pw, pl.ds(CH - STEAL_FF, STEAL_FF)],
                        srcv.at[pl.ds(0, STEAL_FF)])
        pltpu.sync_copy(dst_hbm.at[pw, pl.ds(CH - STEAL_FF, STEAL_FF)],
                        dstv.at[pl.ds(0, STEAL_FF)])
        pltpu.sync_copy(pred_hbm.at[pl.ds(pw * EPW + (CH - STEAL_FF) * K, STEAL_FF * K)],
                        predv.at[pl.ds(0, STEAL_FF * K)])
        pipe(STEAL_FF)

    @pl.when(c == 1)
    def _():
        pipe(CH - STEAL_FF)

    pltpu.sync_copy(ssev, out_hbm.at[c, s])


def _edge_split(idx):
    pad = jnp.full((EP - E,), N, jnp.int32)
    return jnp.concatenate([idx, pad]).reshape(NW, CH, K)


GRID = 8
BR = RP // GRID


def _dinv_of(degp_blk):
    deg = degp_blk[0, :, 0:1] + degp_blk[1, :, 0:1] + 1.0
    return lax.rsqrt(deg)


def _tc_b_body(degp, xp, w1, h1_o, h1s_o):
    dinv = _dinv_of(degp[...])
    h1 = jnp.dot(xp[...], w1[...], preferred_element_type=jnp.float32)
    h1_o[...] = h1
    h1s_o[...] = dinv * h1


def _tc_stage_b(degp, xp, w1):
    return pl.pallas_call(
        _tc_b_body,
        grid=(GRID,),
        in_specs=[
            pl.BlockSpec((NC, BR, DW), lambda i: (0, i, 0)),
            pl.BlockSpec((BR, D), lambda i: (i, 0)),
            pl.BlockSpec((D, H), lambda i: (0, 0)),
        ],
        out_specs=[
            pl.BlockSpec((BR, H), lambda i: (i, 0)),
            pl.BlockSpec((BR, H), lambda i: (i, 0)),
        ],
        out_shape=[
            jax.ShapeDtypeStruct((RP, H), jnp.float32),
            jax.ShapeDtypeStruct((RP, H), jnp.float32),
        ],
    )(degp, xp, w1)


def _tc_c_body(degp, agg1p, h1, w2p, b1, h2_o, h2s_o):
    dinv = _dinv_of(degp[...])
    z1 = jax.nn.relu(dinv * (agg1p[0] + agg1p[1]) + dinv * dinv * h1[...] + b1[...])
    h2 = jnp.dot(z1, w2p[...], preferred_element_type=jnp.float32)
    h2_o[...] = h2
    h2s_o[...] = dinv * h2


def _tc_stage_c(degp, agg1p, h1, w2p, b1):
    return pl.pallas_call(
        _tc_c_body,
        grid=(GRID,),
        in_specs=[
            pl.BlockSpec((NC, BR, DW), lambda i: (0, i, 0)),
            pl.BlockSpec((NC, BR, H), lambda i: (0, i, 0)),
            pl.BlockSpec((BR, H), lambda i: (i, 0)),
            pl.BlockSpec((H, H), lambda i: (0, 0)),
            pl.BlockSpec((1, H), lambda i: (0, 0)),
        ],
        out_specs=[
            pl.BlockSpec((BR, H), lambda i: (i, 0)),
            pl.BlockSpec((BR, H), lambda i: (i, 0)),
        ],
        out_shape=[
            jax.ShapeDtypeStruct((RP, H), jnp.float32),
            jax.ShapeDtypeStruct((RP, H), jnp.float32),
        ],
    )(degp, agg1p, h1, w2p, b1)


def _tc_d_body(degp, agg2p, h2, b2p, fx_o, colsum_o):
    i = pl.program_id(0)
    dinv = _dinv_of(degp[...])
    o2 = dinv * (agg2p[0] + agg2p[1]) + dinv * dinv * h2[...] + b2p[...]
    colmask = lax.broadcasted_iota(jnp.int32, (BR, H), 1) < C
    o2m = jnp.where(colmask, o2, -1e30)
    m = jnp.max(o2m, axis=1, keepdims=True)
    ex = jnp.exp(o2m - m)
    fx = ex / jnp.sum(ex, axis=1, keepdims=True)
    rowmask = (lax.broadcasted_iota(jnp.int32, (BR, H), 0) + i * BR) < N
    fx = jnp.where(rowmask & colmask, fx, 0.0)
    fx_o[...] = fx
    nfx = jnp.log(1.0 - fx * fx)
    cs = jnp.sum(nfx, axis=0)[None, :]
    colsum_o[...] = jnp.pad(cs, ((0, 7), (0, 128 - H)))


def _tc_stage_d(degp, agg2p, h2, b2p):
    return pl.pallas_call(
        _tc_d_body,
        grid=(GRID,),
        in_specs=[
            pl.BlockSpec((NC, BR, DW), lambda i: (0, i, 0)),
            pl.BlockSpec((NC, BR, H), lambda i: (0, i, 0)),
            pl.BlockSpec((BR, H), lambda i: (i, 0)),
            pl.BlockSpec((1, H), lambda i: (0, 0)),
        ],
        out_specs=[
            pl.BlockSpec((BR, H), lambda i: (i, 0)),
            pl.BlockSpec((8, 128), lambda i: (i, 0)),
        ],
        out_shape=[
            jax.ShapeDtypeStruct((RP, H), jnp.float32),
            jax.ShapeDtypeStruct((GRID * 8, 128), jnp.float32),
        ],
    )(degp, agg2p, h2, b2p)


def _tc_e_body(ssep, colsum, loss_o):
    s = jnp.sum(colsum[...], axis=0, keepdims=True)
    cmask = lax.broadcasted_iota(jnp.int32, (1, 128), 1) < C
    preg = -jnp.sum(jnp.where(cmask, jnp.log(1.0001 - jnp.exp(s)), 0.0))
    sse = jnp.sum(ssep[...][:, :, 0:1])
    loss_o[...] = jnp.full((1, 1), sse / E + REG * preg, jnp.float32)


def _tc_stage_e(ssep, colsum):
    return pl.pallas_call(
        _tc_e_body,
        out_shape=jax.ShapeDtypeStruct((1, 1), jnp.float32),
    )(ssep, colsum)


def kernel(x, edge_index, edge_pred, W1, b1, W2, b2):
    src, dst = edge_index[0], edge_index[1]
    src3 = _edge_split(src)
    dst3 = _edge_split(dst)
    zeros_rp = jnp.zeros((RP, H), jnp.float32)
    xp = jnp.pad(x, ((0, RP - N), (0, 0)))
    W2p = jnp.pad(W2, ((0, 0), (0, H - C)))
    b1r = b1.reshape(1, H)
    b2r = jnp.pad(b2, (0, H - C)).reshape(1, H)
    predp = jnp.concatenate([edge_pred, jnp.zeros((EP - E,), jnp.float32)])

    degp = _sc_deg(dst3, jnp.ones((K, DW), jnp.float32),
                   jnp.zeros((RP, DW), jnp.float32))
    h1, h1s = _tc_stage_b(degp, xp, W1)
    agg1p = _sc_agg(h1s, src3, dst3, zeros_rp)
    h2, h2s = _tc_stage_c(degp, agg1p, h1, W2p, b1r)
    agg2p = _sc_agg(h2s, src3, dst3, zeros_rp)
    fxp, colsum = _tc_stage_d(degp, agg2p, h2, b2r)
    ssep = _sc_ff(fxp, src3, dst3, predp)
    loss = _tc_stage_e(ssep, colsum)

    return (fxp[:N, :C], loss[0, 0])


# speedup vs baseline: 1.0623x; 1.0623x over previous
import functools

import jax
import jax.numpy as jnp
from jax import lax
from jax.experimental import pallas as pl
from jax.experimental.pallas import tpu as pltpu, tpu_sc as plsc

N = 10000
E = 320000
D = 128
H = 32
C = 30
REG = 0.01

NC = 2
NS = 16
NW = NC * NS
K = 128
CH = 80
EPW = CH * K
EP = NW * EPW
RPS = 632
RP = NS * RPS
STEAL = 24
STEAL_FF = 40

_MESH = plsc.VectorSubcoreMesh(core_axis_name="c", subcore_axis_name="s")


@functools.partial(
    pl.kernel,
    out_type=jax.ShapeDtypeStruct((NC, RP, H), jnp.float32),
    mesh=_MESH,
    compiler_params=pltpu.CompilerParams(use_tc_tiling_on_sc=False),
    scratch_types=[
        pltpu.VMEM((CH, K), jnp.int32),
        pltpu.VMEM((CH, K), jnp.int32),
        pltpu.VMEM((K, H), jnp.float32),
        pltpu.VMEM((K, H), jnp.float32),
        pltpu.VMEM_SHARED((RP, H), jnp.float32),
        pltpu.SemaphoreType.DMA,
        pltpu.SemaphoreType.DMA,
    ],
)
def _sc_agg(hs_hbm, src_hbm, dst_hbm, zeros_hbm, out_hbm,
            srcv, dstv, rows0, rows1, acc, sem0, sem1):
    c = lax.axis_index("c")
    s = lax.axis_index("s")
    wid = s * NC + c
    row0 = pl.multiple_of(s * RPS, 8)
    pltpu.sync_copy(zeros_hbm.at[pl.ds(row0, RPS)], acc.at[pl.ds(row0, RPS)])
    plsc.subcore_barrier()
    pltpu.sync_copy(src_hbm.at[wid], srcv)
    pltpu.sync_copy(dst_hbm.at[wid], dstv)

    def pipe(cnt):
        pltpu.async_copy(hs_hbm.at[srcv.at[0]], rows0, sem0)
        pltpu.async_copy(hs_hbm.at[srcv.at[1]], rows1, sem1)

        def body(i, carry):
            g = 2 * i
            pltpu.make_async_copy(hs_hbm.at[srcv.at[g]], rows0, sem0).wait()
            pltpu.sync_copy(rows0, acc.at[dstv.at[g]], add=True)

            @pl.when(g + 2 < cnt)
            def _():
                pltpu.async_copy(hs_hbm.at[srcv.at[g + 2]], rows0, sem0)

            pltpu.make_async_copy(hs_hbm.at[srcv.at[g + 1]], rows1, sem1).wait()
            pltpu.sync_copy(rows1, acc.at[dstv.at[g + 1]], add=True)

            @pl.when(g + 3 < cnt)
            def _():
                pltpu.async_copy(hs_hbm.at[srcv.at[g + 3]], rows1, sem1)

            return carry

        lax.fori_loop(0, cnt // 2, body, 0)

    pw = s * NC + 1

    @pl.when(c == 0)
    def _():
        pipe(CH)
        pltpu.sync_copy(src_hbm.at[pw, pl.ds(CH - STEAL, STEAL)], srcv.at[pl.ds(0, STEAL)])
        pltpu.sync_copy(dst_hbm.at[pw, pl.ds(CH - STEAL, STEAL)], dstv.at[pl.ds(0, STEAL)])
        pipe(STEAL)

    @pl.when(c == 1)
    def _():
        pipe(CH - STEAL)

    plsc.subcore_barrier()
    pltpu.sync_copy(acc.at[pl.ds(row0, RPS)], out_hbm.at[c, pl.ds(row0, RPS)])


DW = 16


@functools.partial(
    pl.kernel,
    out_type=jax.ShapeDtypeStruct((NC, RP, DW), jnp.float32),
    mesh=_MESH,
    compiler_params=pltpu.CompilerParams(use_tc_tiling_on_sc=False),
    scratch_types=[
        pltpu.VMEM((CH, K), jnp.int32),
        pltpu.VMEM((K, DW), jnp.float32),
        pltpu.VMEM_SHARED((RP, DW), jnp.float32),
    ],
)
def _sc_deg(dst_hbm, ones_hbm, zeros_hbm, out_hbm, dstv, ones, acc):
    c = lax.axis_index("c")
    s = lax.axis_index("s")
    wid = s * NC + c
    row0 = pl.multiple_of(s * RPS, 8)
    pltpu.sync_copy(zeros_hbm.at[pl.ds(row0, RPS)], acc.at[pl.ds(row0, RPS)])
    pltpu.sync_copy(ones_hbm, ones)
    plsc.subcore_barrier()
    pltpu.sync_copy(dst_hbm.at[wid], dstv)

    def body(ci, carry):
        pltpu.sync_copy(ones, acc.at[dstv.at[ci]], add=True)
        return carry

    lax.fori_loop(0, CH, body, 0)
    plsc.subcore_barrier()
    pltpu.sync_copy(acc.at[pl.ds(row0, RPS)], out_hbm.at[c, pl.ds(row0, RPS)])


@functools.partial(
    pl.kernel,
    out_type=jax.ShapeDtypeStruct((NC, NS, 16), jnp.float32),
    mesh=_MESH,
    compiler_params=pltpu.CompilerParams(
        use_tc_tiling_on_sc=False, needs_layout_passes=False),
    scratch_types=[
        pltpu.VMEM((CH, K), jnp.int32),
        pltpu.VMEM((CH, K), jnp.int32),
        pltpu.VMEM((CH * K,), jnp.float32),
        pltpu.VMEM((K, H), jnp.float32),
        pltpu.VMEM((K, H), jnp.float32),
        pltpu.VMEM((K, H), jnp.float32),
        pltpu.VMEM((K, H), jnp.float32),
        pltpu.VMEM((16,), jnp.float32),
        pltpu.SemaphoreType.DMA,
        pltpu.SemaphoreType.DMA,
        pltpu.SemaphoreType.DMA,
        pltpu.SemaphoreType.DMA,
    ],
)
def _sc_ff(fx_hbm, src_hbm, dst_hbm, pred_hbm, out_hbm,
           srcv, dstv, predv, rs0, rd0, rs1, rd1, ssev,
           semS0, semD0, semS1, semD1):
    c = lax.axis_index("c")
    s = lax.axis_index("s")
    wid = s * NC + c
    pltpu.sync_copy(src_hbm.at[wid], srcv)
    pltpu.sync_copy(dst_hbm.at[wid], dstv)
    pltpu.sync_copy(pred_hbm.at[pl.ds(wid * EPW, EPW)], predv)

    def start(ci, rows_s, rows_d, sem_s, sem_d):
        pltpu.async_copy(fx_hbm.at[srcv.at[ci]], rows_s, sem_s)
        pltpu.async_copy(fx_hbm.at[dstv.at[ci]], rows_d, sem_d)

    def wait(ci, rows_s, rows_d, sem_s, sem_d):
        pltpu.make_async_copy(fx_hbm.at[srcv.at[ci]], rows_s, sem_s).wait()
        pltpu.make_async_copy(fx_hbm.at[dstv.at[ci]], rows_d, sem_d).wait()

    def chunk_dot(ci, rows_s, rows_d, sse):
        accs = [sse, 0.0, 0.0, 0.0]
        for e0 in range(0, K, 16):
            pv = predv[pl.ds(ci * K + e0, 16)]
            for j in range(16):
                e = e0 + j
                a0 = rows_s[e, pl.ds(0, 16)]
                a1 = rows_s[e, pl.ds(16, 16)]
                b0 = rows_d[e, pl.ds(0, 16)]
                b1 = rows_d[e, pl.ds(16, 16)]
                dot = jnp.sum(a0 * b0 + a1 * b1)
                err = dot - pv[j]
                accs[e % 4] += err * err
        return (accs[0] + accs[1]) + (accs[2] + accs[3])

    def pipe(cnt):
        start(0, rs0, rd0, semS0, semD0)
        start(1, rs1, rd1, semS1, semD1)

        def body(i, sse):
            g = 2 * i
            wait(g, rs0, rd0, semS0, semD0)
            sse = chunk_dot(g, rs0, rd0, sse)

            @pl.when(g + 2 < cnt)
            def _():
                start(g + 2, rs0, rd0, semS0, semD0)

            wait(g + 1, rs1, rd1, semS1, semD1)
            sse = chunk_dot(g + 1, rs1, rd1, sse)

            @pl.when(g + 3 < cnt)
            def _():
                start(g + 3, rs1, rd1, semS1, semD1)

            return sse

        sse = lax.fori_loop(0, cnt // 2, body, jnp.float32(0.0))
        ssev[...] = ssev[...] + jnp.full((16,), sse, jnp.float32)

    ssev[...] = jnp.zeros((16,), jnp.float32)
    pw = s * NC + 1

    @pl.when(c == 0)
    def _():
        pipe(CH)
        pltpu.sync_copy(src_hbm.at[pw, pl.ds(CH - STEAL_FF, STEAL_FF)],
                        srcv.at[pl.ds(0, STEAL_FF)])
        pltpu.sync_copy(dst_hbm.at[pw, pl.ds(CH - STEAL_FF, STEAL_FF)],
                        dstv.at[pl.ds(0, STEAL_FF)])
        pltpu.sync_copy(pred_hbm.at[pl.ds(pw * EPW + (CH - STEAL_FF) * K, STEAL_FF * K)],
                        predv.at[pl.ds(0, STEAL_FF * K)])
        pipe(STEAL_FF)

    @pl.when(c == 1)
    def _():
        pipe(CH - STEAL_FF)

    pltpu.sync_copy(ssev, out_hbm.at[c, s])


def _edge_split(idx):
    pad = jnp.full((EP - E,), N, jnp.int32)
    return jnp.concatenate([idx, pad]).reshape(NW, CH, K)


GRID = 8
BR = RP // GRID


def _dinv_of(degp_blk):
    deg = degp_blk[0, :, 0:1] + degp_blk[1, :, 0:1] + 1.0
    return lax.rsqrt(deg)


def _tc_b_body(degp, xp, w1, h1_o, h1s_o):
    dinv = _dinv_of(degp[...])
    h1 = jnp.dot(xp[...], w1[...], preferred_element_type=jnp.float32)
    h1_o[...] = h1
    h1s_o[...] = dinv * h1


def _tc_stage_b(degp, xp, w1):
    return pl.pallas_call(
        _tc_b_body,
        grid=(GRID,),
        in_specs=[
            pl.BlockSpec((NC, BR, DW), lambda i: (0, i, 0)),
            pl.BlockSpec((BR, D), lambda i: (i, 0)),
            pl.BlockSpec((D, H), lambda i: (0, 0)),
        ],
        out_specs=[
            pl.BlockSpec((BR, H), lambda i: (i, 0)),
            pl.BlockSpec((BR, H), lambda i: (i, 0)),
        ],
        out_shape=[
            jax.ShapeDtypeStruct((RP, H), jnp.float32),
            jax.ShapeDtypeStruct((RP, H), jnp.float32),
        ],
    )(degp, xp, w1)


def _tc_c_body(degp, agg1p, h1, w2p, b1, h2_o, h2s_o):
    dinv = _dinv_of(degp[...])
    z1 = jax.nn.relu(dinv * (agg1p[0] + agg1p[1]) + dinv * dinv * h1[...] + b1[...])
    h2 = jnp.dot(z1, w2p[...], preferred_element_type=jnp.float32)
    h2_o[...] = h2
    h2s_o[...] = dinv * h2


def _tc_stage_c(degp, agg1p, h1, w2p, b1):
    return pl.pallas_call(
        _tc_c_body,
        grid=(GRID,),
        in_specs=[
            pl.BlockSpec((NC, BR, DW), lambda i: (0, i, 0)),
            pl.BlockSpec((NC, BR, H), lambda i: (0, i, 0)),
            pl.BlockSpec((BR, H), lambda i: (i, 0)),
            pl.BlockSpec((H, H), lambda i: (0, 0)),
            pl.BlockSpec((1, H), lambda i: (0, 0)),
        ],
        out_specs=[
            pl.BlockSpec((BR, H), lambda i: (i, 0)),
            pl.BlockSpec((BR, H), lambda i: (i, 0)),
        ],
        out_shape=[
            jax.ShapeDtypeStruct((RP, H), jnp.float32),
            jax.ShapeDtypeStruct((RP, H), jnp.float32),
        ],
    )(degp, agg1p, h1, w2p, b1)


def _tc_d_body(degp, agg2p, h2, b2p, fx_o, colsum_o):
    i = pl.program_id(0)
    dinv = _dinv_of(degp[...])
    o2 = dinv * (agg2p[0] + agg2p[1]) + dinv * dinv * h2[...] + b2p[...]
    colmask = lax.broadcasted_iota(jnp.int32, (BR, H), 1) < C
    o2m = jnp.where(colmask, o2, -1e30)
    m = jnp.max(o2m, axis=1, keepdims=True)
    ex = jnp.exp(o2m - m)
    fx = ex / jnp.sum(ex, axis=1, keepdims=True)
    rowmask = (lax.broadcasted_iota(jnp.int32, (BR, H), 0) + i * BR) < N
    fx = jnp.where(rowmask & colmask, fx, 0.0)
    fx_o[...] = fx
    nfx = jnp.log(1.0 - fx * fx)
    cs = jnp.sum(nfx, axis=0)[None, :]
    colsum_o[...] = jnp.pad(cs, ((0, 7), (0, 128 - H)))


def _tc_stage_d(degp, agg2p, h2, b2p):
    return pl.pallas_call(
        _tc_d_body,
        grid=(GRID,),
        in_specs=[
            pl.BlockSpec((NC, BR, DW), lambda i: (0, i, 0)),
            pl.BlockSpec((NC, BR, H), lambda i: (0, i, 0)),
            pl.BlockSpec((BR, H), lambda i: (i, 0)),
            pl.BlockSpec((1, H), lambda i: (0, 0)),
        ],
        out_specs=[
            pl.BlockSpec((BR, H), lambda i: (i, 0)),
            pl.BlockSpec((8, 128), lambda i: (i, 0)),
        ],
        out_shape=[
            jax.ShapeDtypeStruct((RP, H), jnp.float32),
            jax.ShapeDtypeStruct((GRID * 8, 128), jnp.float32),
        ],
    )(degp, agg2p, h2, b2p)


def _tc_e_body(ssep, colsum, loss_o):
    s = jnp.sum(colsum[...], axis=0, keepdims=True)
    cmask = lax.broadcasted_iota(jnp.int32, (1, 128), 1) < C
    preg = -jnp.sum(jnp.where(cmask, jnp.log(1.0001 - jnp.exp(s)), 0.0))
    sse = jnp.sum(ssep[...][:, :, 0:1])
    loss_o[...] = jnp.full((1, 1), sse / E + REG * preg, jnp.float32)


def _tc_stage_e(ssep, colsum):
    return pl.pallas_call(
        _tc_e_body,
        out_shape=jax.ShapeDtypeStruct((1, 1), jnp.float32),
    )(ssep, colsum)


def kernel(x, edge_index, edge_pred, W1, b1, W2, b2):
    src, dst = edge_index[0], edge_index[1]
    src3 = _edge_split(src)
    dst3 = _edge_split(dst)
    zeros_rp = jnp.zeros((RP, H), jnp.float32)
    xp = jnp.pad(x, ((0, RP - N), (0, 0)))
    W2p = jnp.pad(W2, ((0, 0), (0, H - C)))
    b1r = b1.reshape(1, H)
    b2r = jnp.pad(b2, (0, H - C)).reshape(1, H)
    predp = jnp.concatenate([edge_pred, jnp.zeros((EP - E,), jnp.float32)])

    degp = _sc_deg(dst3, jnp.ones((K, DW), jnp.float32),
                   jnp.zeros((RP, DW), jnp.float32))
    h1, h1s = _tc_stage_b(degp, xp, W1)
    agg1p = _sc_agg(h1s, src3, dst3, zeros_rp)
    h2, h2s = _tc_stage_c(degp, agg1p, h1, W2p, b1r)
    agg2p = _sc_agg(h2s, src3, dst3, zeros_rp)
    fxp, colsum = _tc_stage_d(degp, agg2p, h2, b2r)
    ssep = _sc_ff(fxp, src3, dst3, predp)
    loss = _tc_stage_e(ssep, colsum)

    return (fxp[:N, :C], loss[0, 0])


# speedup vs baseline: 1.6915x; 1.5922x over previous
import functools

import jax
import jax.numpy as jnp
from jax import lax
from jax.experimental import pallas as pl
from jax.experimental.pallas import tpu as pltpu, tpu_sc as plsc

N = 10000
E = 320000
D = 128
H = 32
C = 30
REG = 0.01

NC = 2
NS = 16
K = 128
TOT = E // K
F0 = 104
F1 = 52
B1 = NS * F0
CP1 = 53
RPS = 632
RP = NS * RPS

_MESH = plsc.VectorSubcoreMesh(core_axis_name="c", subcore_axis_name="s")


def _alloc(c, s):
    base = jnp.where(c == 0, s * F0, B1 + F1 * s + jnp.maximum(0, s - 12))
    cnt = jnp.where(c == 0, F0, F1 + jnp.where(s >= 12, 1, 0))
    return base, cnt


@functools.partial(
    pl.kernel,
    out_type=jax.ShapeDtypeStruct((NC, RP, H), jnp.float32),
    mesh=_MESH,
    compiler_params=pltpu.CompilerParams(use_tc_tiling_on_sc=False),
    scratch_types=[
        pltpu.VMEM((F0, K), jnp.int32),
        pltpu.VMEM((F0, K), jnp.int32),
        pltpu.VMEM((K, H), jnp.float32),
        pltpu.VMEM((K, H), jnp.float32),
        pltpu.VMEM_SHARED((RP, H), jnp.float32),
        pltpu.SemaphoreType.DMA,
        pltpu.SemaphoreType.DMA,
    ],
)
def _sc_agg(hs_hbm, ei_hbm, zeros_hbm, out_hbm,
            srcv, dstv, rows0, rows1, acc, sem0, sem1):
    c = lax.axis_index("c")
    s = lax.axis_index("s")
    row0 = pl.multiple_of(s * RPS, 8)
    pltpu.sync_copy(zeros_hbm.at[pl.ds(row0, RPS)], acc.at[pl.ds(row0, RPS)])
    plsc.subcore_barrier()
    base, cnt = _alloc(c, s)

    @pl.when(c == 0)
    def _():
        pltpu.sync_copy(ei_hbm.at[0, pl.ds(base, F0)], srcv)
        pltpu.sync_copy(ei_hbm.at[1, pl.ds(base, F0)], dstv)

    @pl.when(c == 1)
    def _():
        pltpu.sync_copy(ei_hbm.at[0, pl.ds(base, CP1)], srcv.at[pl.ds(0, CP1)])
        pltpu.sync_copy(ei_hbm.at[1, pl.ds(base, CP1)], dstv.at[pl.ds(0, CP1)])

    pltpu.async_copy(hs_hbm.at[srcv.at[0]], rows0, sem0)
    pltpu.async_copy(hs_hbm.at[srcv.at[1]], rows1, sem1)

    def body(i, carry):
        g = 2 * i
        pltpu.make_async_copy(hs_hbm.at[srcv.at[g]], rows0, sem0).wait()
        pltpu.sync_copy(rows0, acc.at[dstv.at[g]], add=True)

        @pl.when(g + 2 < cnt)
        def _():
            pltpu.async_copy(hs_hbm.at[srcv.at[g + 2]], rows0, sem0)

        pltpu.make_async_copy(hs_hbm.at[srcv.at[g + 1]], rows1, sem1).wait()
        pltpu.sync_copy(rows1, acc.at[dstv.at[g + 1]], add=True)

        @pl.when(g + 3 < cnt)
        def _():
            pltpu.async_copy(hs_hbm.at[srcv.at[g + 3]], rows1, sem1)

        return carry

    lax.fori_loop(0, cnt // 2, body, 0)

    @pl.when(cnt % 2 == 1)
    def _():
        pltpu.make_async_copy(hs_hbm.at[srcv.at[cnt - 1]], rows0, sem0).wait()
        pltpu.sync_copy(rows0, acc.at[dstv.at[cnt - 1]], add=True)

    plsc.subcore_barrier()
    pltpu.sync_copy(acc.at[pl.ds(row0, RPS)], out_hbm.at[c, pl.ds(row0, RPS)])


DW = 16
FD = 78
CPD = 79


@functools.partial(
    pl.kernel,
    out_type=jax.ShapeDtypeStruct((NC, RP, DW), jnp.float32),
    mesh=_MESH,
    compiler_params=pltpu.CompilerParams(use_tc_tiling_on_sc=False),
    scratch_types=[
        pltpu.VMEM((CPD, K), jnp.int32),
        pltpu.VMEM((K, DW), jnp.float32),
        pltpu.VMEM_SHARED((RP, DW), jnp.float32),
    ],
)
def _sc_deg(ei_hbm, ones_hbm, zeros_hbm, out_hbm, dstv, ones, acc):
    c = lax.axis_index("c")
    s = lax.axis_index("s")
    wid = s * NC + c
    row0 = pl.multiple_of(s * RPS, 8)
    pltpu.sync_copy(zeros_hbm.at[pl.ds(row0, RPS)], acc.at[pl.ds(row0, RPS)])
    pltpu.sync_copy(ones_hbm, ones)
    plsc.subcore_barrier()
    base = FD * wid + jnp.maximum(0, wid - 28)
    cnt = FD + jnp.where(wid >= 28, 1, 0)
    pltpu.sync_copy(ei_hbm.at[1, pl.ds(base, CPD)], dstv)

    def body(ci, carry):
        pltpu.sync_copy(ones, acc.at[dstv.at[ci]], add=True)
        return carry

    lax.fori_loop(0, cnt, body, 0)
    plsc.subcore_barrier()
    pltpu.sync_copy(acc.at[pl.ds(row0, RPS)], out_hbm.at[c, pl.ds(row0, RPS)])


@functools.partial(
    pl.kernel,
    out_type=jax.ShapeDtypeStruct((NC, NS, 16), jnp.float32),
    mesh=_MESH,
    compiler_params=pltpu.CompilerParams(
        use_tc_tiling_on_sc=False, needs_layout_passes=False),
    scratch_types=[
        pltpu.VMEM((F0, K), jnp.int32),
        pltpu.VMEM((F0, K), jnp.int32),
        pltpu.VMEM((F0, K), jnp.float32),
        pltpu.VMEM((K, H), jnp.float32),
        pltpu.VMEM((K, H), jnp.float32),
        pltpu.VMEM((K, H), jnp.float32),
        pltpu.VMEM((K, H), jnp.float32),
        pltpu.VMEM((16,), jnp.float32),
        pltpu.SemaphoreType.DMA,
        pltpu.SemaphoreType.DMA,
        pltpu.SemaphoreType.DMA,
        pltpu.SemaphoreType.DMA,
    ],
)
def _sc_ff(fx_hbm, ei_hbm, pred_hbm, out_hbm,
           srcv, dstv, predv, rs0, rd0, rs1, rd1, ssev,
           semS0, semD0, semS1, semD1):
    c = lax.axis_index("c")
    s = lax.axis_index("s")
    base, cnt = _alloc(c, s)

    @pl.when(c == 0)
    def _():
        pltpu.sync_copy(ei_hbm.at[0, pl.ds(base, F0)], srcv)
        pltpu.sync_copy(ei_hbm.at[1, pl.ds(base, F0)], dstv)
        pltpu.sync_copy(pred_hbm.at[pl.ds(base, F0)], predv)

    @pl.when(c == 1)
    def _():
        pltpu.sync_copy(ei_hbm.at[0, pl.ds(base, CP1)], srcv.at[pl.ds(0, CP1)])
        pltpu.sync_copy(ei_hbm.at[1, pl.ds(base, CP1)], dstv.at[pl.ds(0, CP1)])
        pltpu.sync_copy(pred_hbm.at[pl.ds(base, CP1)], predv.at[pl.ds(0, CP1)])

    def start(ci, rows_s, rows_d, sem_s, sem_d):
        pltpu.async_copy(fx_hbm.at[srcv.at[ci]], rows_s, sem_s)
        pltpu.async_copy(fx_hbm.at[dstv.at[ci]], rows_d, sem_d)

    def wait(ci, rows_s, rows_d, sem_s, sem_d):
        pltpu.make_async_copy(fx_hbm.at[srcv.at[ci]], rows_s, sem_s).wait()
        pltpu.make_async_copy(fx_hbm.at[dstv.at[ci]], rows_d, sem_d).wait()

    def chunk_dot(ci, rows_s, rows_d, sse):
        accs = [sse, 0.0, 0.0, 0.0]
        for e0 in range(0, K, 16):
            pv = predv[ci, pl.ds(e0, 16)]
            for j in range(16):
                e = e0 + j
                a0 = rows_s[e, pl.ds(0, 16)]
                a1 = rows_s[e, pl.ds(16, 16)]
                b0 = rows_d[e, pl.ds(0, 16)]
                b1 = rows_d[e, pl.ds(16, 16)]
                dot = jnp.sum(a0 * b0 + a1 * b1)
                err = dot - pv[j]
                accs[e % 4] += err * err
        return (accs[0] + accs[1]) + (accs[2] + accs[3])

    start(0, rs0, rd0, semS0, semD0)
    start(1, rs1, rd1, semS1, semD1)

    def body(i, sse):
        g = 2 * i
        wait(g, rs0, rd0, semS0, semD0)
        sse = chunk_dot(g, rs0, rd0, sse)

        @pl.when(g + 2 < cnt)
        def _():
            start(g + 2, rs0, rd0, semS0, semD0)

        wait(g + 1, rs1, rd1, semS1, semD1)
        sse = chunk_dot(g + 1, rs1, rd1, sse)

        @pl.when(g + 3 < cnt)
        def _():
            start(g + 3, rs1, rd1, semS1, semD1)

        return sse

    sse = lax.fori_loop(0, cnt // 2, body, jnp.float32(0.0))
    ssev[...] = jnp.full((16,), sse, jnp.float32)

    @pl.when(cnt % 2 == 1)
    def _():
        wait(cnt - 1, rs0, rd0, semS0, semD0)
        tail = chunk_dot(cnt - 1, rs0, rd0, jnp.float32(0.0))
        ssev[...] = ssev[...] + jnp.full((16,), tail, jnp.float32)

    pltpu.sync_copy(ssev, out_hbm.at[c, s])


GRID = 8
BR = RP // GRID


def _dinv_of(degp_blk):
    deg = degp_blk[0, :, 0:1] + degp_blk[1, :, 0:1] + 1.0
    return lax.rsqrt(deg)


def _tc_b_body(degp, xp, w1, h1_o, h1s_o):
    dinv = _dinv_of(degp[...])
    h1 = jnp.dot(xp[...], w1[...], preferred_element_type=jnp.float32)
    h1_o[...] = h1
    h1s_o[...] = dinv * h1


def _tc_stage_b(degp, xp, w1):
    return pl.pallas_call(
        _tc_b_body,
        grid=(GRID,),
        in_specs=[
            pl.BlockSpec((NC, BR, DW), lambda i: (0, i, 0)),
            pl.BlockSpec((BR, D), lambda i: (i, 0)),
            pl.BlockSpec((D, H), lambda i: (0, 0)),
        ],
        out_specs=[
            pl.BlockSpec((BR, H), lambda i: (i, 0)),
            pl.BlockSpec((BR, H), lambda i: (i, 0)),
        ],
        out_shape=[
            jax.ShapeDtypeStruct((RP, H), jnp.float32),
            jax.ShapeDtypeStruct((RP, H), jnp.float32),
        ],
    )(degp, xp, w1)


def _tc_c_body(degp, agg1p, h1, w2p, b1, h2_o, h2s_o):
    dinv = _dinv_of(degp[...])
    z1 = jax.nn.relu(dinv * (agg1p[0] + agg1p[1]) + dinv * dinv * h1[...] + b1[...])
    h2 = jnp.dot(z1, w2p[...], preferred_element_type=jnp.float32)
    h2_o[...] = h2
    h2s_o[...] = dinv * h2


def _tc_stage_c(degp, agg1p, h1, w2p, b1):
    return pl.pallas_call(
        _tc_c_body,
        grid=(GRID,),
        in_specs=[
            pl.BlockSpec((NC, BR, DW), lambda i: (0, i, 0)),
            pl.BlockSpec((NC, BR, H), lambda i: (0, i, 0)),
            pl.BlockSpec((BR, H), lambda i: (i, 0)),
            pl.BlockSpec((H, H), lambda i: (0, 0)),
            pl.BlockSpec((1, H), lambda i: (0, 0)),
        ],
        out_specs=[
            pl.BlockSpec((BR, H), lambda i: (i, 0)),
            pl.BlockSpec((BR, H), lambda i: (i, 0)),
        ],
        out_shape=[
            jax.ShapeDtypeStruct((RP, H), jnp.float32),
            jax.ShapeDtypeStruct((RP, H), jnp.float32),
        ],
    )(degp, agg1p, h1, w2p, b1)


def _tc_d_body(degp, agg2p, h2, b2p, fx_o, colsum_o):
    i = pl.program_id(0)
    dinv = _dinv_of(degp[...])
    o2 = dinv * (agg2p[0] + agg2p[1]) + dinv * dinv * h2[...] + b2p[...]
    colmask = lax.broadcasted_iota(jnp.int32, (BR, H), 1) < C
    o2m = jnp.where(colmask, o2, -1e30)
    m = jnp.max(o2m, axis=1, keepdims=True)
    ex = jnp.exp(o2m - m)
    fx = ex / jnp.sum(ex, axis=1, keepdims=True)
    rowmask = (lax.broadcasted_iota(jnp.int32, (BR, H), 0) + i * BR) < N
    fx = jnp.where(rowmask & colmask, fx, 0.0)
    fx_o[...] = fx
    nfx = jnp.log(1.0 - fx * fx)
    cs = jnp.sum(nfx, axis=0)[None, :]
    colsum_o[...] = jnp.pad(cs, ((0, 7), (0, 128 - H)))


def _tc_stage_d(degp, agg2p, h2, b2p):
    return pl.pallas_call(
        _tc_d_body,
        grid=(GRID,),
        in_specs=[
            pl.BlockSpec((NC, BR, DW), lambda i: (0, i, 0)),
            pl.BlockSpec((NC, BR, H), lambda i: (0, i, 0)),
            pl.BlockSpec((BR, H), lambda i: (i, 0)),
            pl.BlockSpec((1, H), lambda i: (0, 0)),
        ],
        out_specs=[
            pl.BlockSpec((BR, H), lambda i: (i, 0)),
            pl.BlockSpec((8, 128), lambda i: (i, 0)),
        ],
        out_shape=[
            jax.ShapeDtypeStruct((RP, H), jnp.float32),
            jax.ShapeDtypeStruct((GRID * 8, 128), jnp.float32),
        ],
    )(degp, agg2p, h2, b2p)


def _tc_e_body(ssep, colsum, loss_o):
    s = jnp.sum(colsum[...], axis=0, keepdims=True)
    cmask = lax.broadcasted_iota(jnp.int32, (1, 128), 1) < C
    preg = -jnp.sum(jnp.where(cmask, jnp.log(1.0001 - jnp.exp(s)), 0.0))
    sse = jnp.sum(ssep[...][:, :, 0:1])
    loss_o[...] = jnp.full((1, 1), sse / E + REG * preg, jnp.float32)


def _tc_stage_e(ssep, colsum):
    return pl.pallas_call(
        _tc_e_body,
        out_shape=jax.ShapeDtypeStruct((1, 1), jnp.float32),
    )(ssep, colsum)


def kernel(x, edge_index, edge_pred, W1, b1, W2, b2):
    ei3 = edge_index.reshape(2, TOT, K)
    predr = edge_pred.reshape(TOT, K)
    zeros_rp = jnp.zeros((RP, H), jnp.float32)
    xp = jnp.pad(x, ((0, RP - N), (0, 0)))
    W2p = jnp.pad(W2, ((0, 0), (0, H - C)))
    b1r = b1.reshape(1, H)
    b2r = jnp.pad(b2, (0, H - C)).reshape(1, H)

    degp = _sc_deg(ei3, jnp.ones((K, DW), jnp.float32),
                   jnp.zeros((RP, DW), jnp.float32))
    h1, h1s = _tc_stage_b(degp, xp, W1)
    agg1p = _sc_agg(h1s, ei3, zeros_rp)
    h2, h2s = _tc_stage_c(degp, agg1p, h1, W2p, b1r)
    agg2p = _sc_agg(h2s, ei3, zeros_rp)
    fxp, colsum = _tc_stage_d(degp, agg2p, h2, b2r)
    ssep = _sc_ff(fxp, ei3, predr)
    loss = _tc_stage_e(ssep, colsum)

    return (fxp[:N, :C], loss[0, 0])
